# baseline (speedup 1.0000x reference)
import jax
import jax.numpy as jnp
from jax import lax
from jax.experimental import pallas as pl
from jax.experimental.pallas import tpu as pltpu
from jax.experimental.pallas import tpu_sc as plsc

NC = 2
NS = 16
B = 256
C, H, W = 4, 64, 64
P = C * H
PBLK = 16


def _gather_body(t_hbm, ac_hbm, am_hbm, a_out, am_out, t_v, a_v, am_v, csem):
    wid = lax.axis_index("s") * NC + lax.axis_index("c")

    @pl.when(wid == 0)
    def _():
        pltpu.sync_copy(t_hbm, t_v)
        ccps = []
        for h in range(2):
            sl = pl.ds(h * 128, 128)
            ccps.append(pltpu.async_copy(
                ac_hbm.at[t_v.at[sl]], a_v.at[sl], csem))
            ccps.append(pltpu.async_copy(
                am_hbm.at[t_v.at[sl]], am_v.at[sl], csem))
        for cp in ccps:
            cp.wait()
        pltpu.sync_copy(a_v, a_out)
        pltpu.sync_copy(am_v, am_out)


def _sc_gather(t32, ac, am):
    mesh = plsc.VectorSubcoreMesh(
        core_axis_name="c", subcore_axis_name="s",
        num_cores=NC, num_subcores=NS)
    f = pl.kernel(
        _gather_body,
        out_type=(jax.ShapeDtypeStruct((B,), jnp.float32),
                  jax.ShapeDtypeStruct((B,), jnp.float32)),
        mesh=mesh,
        compiler_params=pltpu.CompilerParams(skip_device_barrier=True),
        scratch_types=[
            pltpu.VMEM((B,), jnp.int32),
            pltpu.VMEM((B,), jnp.float32),
            pltpu.VMEM((B,), jnp.float32),
            pltpu.SemaphoreType.DMA,
        ],
    )
    return f(t32, ac, am)


def _dense_body(a_ref, am_ref, x_ref, n_ref, o_ref, no_ref):
    n = n_ref[...]
    o_ref[...] = a_ref[...] * x_ref[...] + am_ref[...] * n
    no_ref[...] = n


def _tc_dense(a2, am2, x0T, nT):
    blk = pl.BlockSpec((PBLK, W, B), lambda i: (i, 0, 0))
    cblk = pl.BlockSpec((1, 1, B), lambda i: (0, 0, 0))
    return pl.pallas_call(
        _dense_body,
        grid=(P // PBLK,),
        in_specs=[cblk, cblk, blk, blk],
        out_specs=(blk, blk),
        out_shape=(jax.ShapeDtypeStruct((P, W, B), jnp.float32),
                   jax.ShapeDtypeStruct((P, W, B), jnp.float32)),
    )(a2, am2, x0T, nT)


@jax.jit
def _run(x0, t32, noise, ac, am):
    x0T = x0.transpose(1, 2, 3, 0).reshape(P, W, B)
    nT = noise.transpose(1, 2, 3, 0).reshape(P, W, B)
    a_all, am_all = _sc_gather(t32, ac, am)
    outT, noutT = _tc_dense(a_all.reshape(1, 1, B), am_all.reshape(1, 1, B),
                            x0T, nT)
    out = outT.reshape(C, H, W, B).transpose(3, 0, 1, 2)
    nout = noutT.reshape(C, H, W, B).transpose(3, 0, 1, 2)
    return out, nout


def kernel(x0, t, noise, sqrt_ac, sqrt_1mac):
    return _run(x0, t.astype(jnp.int32), noise, sqrt_ac, sqrt_1mac)


# speedup vs baseline: 1.0318x; 1.0318x over previous
import jax
import jax.numpy as jnp
from jax import lax
from jax.experimental import pallas as pl
from jax.experimental.pallas import tpu as pltpu
from jax.experimental.pallas import tpu_sc as plsc

NC = 2
NS = 16
B = 256
C, H, W = 4, 64, 64
P = C * H
PBLK = 16


def _gather_body(t_hbm, ac_hbm, am_hbm, a_out, am_out, t_v, a_v, am_v, csem):
    wid = lax.axis_index("s")

    @pl.when(wid == 0)
    def _():
        pltpu.sync_copy(t_hbm, t_v)
        ccps = []
        for h in range(2):
            sl = pl.ds(h * 128, 128)
            ccps.append(pltpu.async_copy(
                ac_hbm.at[t_v.at[sl]], a_v.at[sl], csem))
            ccps.append(pltpu.async_copy(
                am_hbm.at[t_v.at[sl]], am_v.at[sl], csem))
        for cp in ccps:
            cp.wait()
        cpo = pltpu.async_copy(a_v, a_out, csem)
        cpm = pltpu.async_copy(am_v, am_out, csem)
        cpo.wait()
        cpm.wait()


def _sc_gather(t32, ac, am):
    mesh = plsc.VectorSubcoreMesh(
        core_axis_name="c", subcore_axis_name="s",
        num_cores=1, num_subcores=NS)
    f = pl.kernel(
        _gather_body,
        out_type=(jax.ShapeDtypeStruct((B,), jnp.float32),
                  jax.ShapeDtypeStruct((B,), jnp.float32)),
        mesh=mesh,
        scratch_types=[
            pltpu.VMEM((B,), jnp.int32),
            pltpu.VMEM((B,), jnp.float32),
            pltpu.VMEM((B,), jnp.float32),
            pltpu.SemaphoreType.DMA,
        ],
    )
    return f(t32, ac, am)


def _dense_body(a_ref, am_ref, x_ref, n_ref, o_ref, no_ref):
    n = n_ref[...]
    o_ref[...] = a_ref[...] * x_ref[...] + am_ref[...] * n
    no_ref[...] = n


def _tc_dense(a2, am2, x0T, nT):
    blk = pl.BlockSpec((PBLK, W, B), lambda i: (i, 0, 0))
    cblk = pl.BlockSpec((1, 1, B), lambda i: (0, 0, 0))
    return pl.pallas_call(
        _dense_body,
        grid=(P // PBLK,),
        in_specs=[cblk, cblk, blk, blk],
        out_specs=(blk, blk),
        out_shape=(jax.ShapeDtypeStruct((P, W, B), jnp.float32),
                   jax.ShapeDtypeStruct((P, W, B), jnp.float32)),
    )(a2, am2, x0T, nT)


@jax.jit
def _run(x0, t32, noise, ac, am):
    x0T = x0.transpose(1, 2, 3, 0).reshape(P, W, B)
    nT = noise.transpose(1, 2, 3, 0).reshape(P, W, B)
    a_all, am_all = _sc_gather(t32, ac, am)
    outT, noutT = _tc_dense(a_all.reshape(1, 1, B), am_all.reshape(1, 1, B),
                            x0T, nT)
    out = outT.reshape(C, H, W, B).transpose(3, 0, 1, 2)
    nout = noutT.reshape(C, H, W, B).transpose(3, 0, 1, 2)
    return out, nout


def kernel(x0, t, noise, sqrt_ac, sqrt_1mac):
    return _run(x0, t.astype(jnp.int32), noise, sqrt_ac, sqrt_1mac)
